# baseline (speedup 1.0000x reference)
import functools

import jax
import jax.numpy as jnp
from jax import lax
from jax.experimental import pallas as pl
from jax.experimental.pallas import tpu as pltpu
from jax.experimental.pallas import tpu_sc as plsc

B, T, D = 16, 4096, 1024
NC = 2
BPC = B // NC


def _last_row_gather(x_flat, seq_len):
    mesh = plsc.ScalarSubcoreMesh(axis_name="c", num_cores=NC)

    @functools.partial(
        pl.kernel,
        mesh=mesh,
        out_type=jax.ShapeDtypeStruct((B, D), jnp.float32),
        scratch_types=[
            pltpu.SMEM((B,), jnp.int32),
            pltpu.SemaphoreType.DMA,
        ],
    )
    def k(x_hbm, seq_hbm, out_hbm, seq_s, sem):
        c = lax.axis_index("c")
        pltpu.sync_copy(seq_hbm, seq_s)
        copies = []
        for i in range(BPC):
            b = c * BPC + i
            row = b * T + seq_s[b] - 1
            copies.append(
                pltpu.make_async_copy(
                    x_hbm.at[pl.ds(row, 1)], out_hbm.at[pl.ds(b, 1)], sem
                )
            )
            copies[-1].start()
        for cp in copies:
            cp.wait()

    return k(x_flat, seq_len)


def kernel(x, seq_len):
    return _last_row_gather(x.reshape(B * T, D), seq_len.astype(jnp.int32))
